# baseline (speedup 1.0000x reference)
import functools

import jax
import jax.numpy as jnp
from jax import lax
from jax.experimental import pallas as pl
from jax.experimental.pallas import tpu as pltpu
from jax.experimental.pallas import tpu_sc as plsc

EMBED_DIM = 128
NUM_CORES = 2
NUM_SUBCORES = 16
NUM_WORKERS = NUM_CORES * NUM_SUBCORES
HIST = 50
HIST_PAD = 56
ITEMS_PER_STEP = 2
CHUNK = ITEMS_PER_STEP * HIST_PAD
NBUF = 4
LAG = 2


@functools.lru_cache(maxsize=None)
def _make_kernel(batch, dim):
    items_per_w = batch // NUM_WORKERS
    n_steps = items_per_w // ITEMS_PER_STEP
    assert n_steps % NBUF == 0 and n_steps >= 2 * NBUF
    n_groups = n_steps // NBUF
    mesh = plsc.VectorSubcoreMesh(core_axis_name="c", subcore_axis_name="s")

    @functools.partial(
        pl.kernel,
        mesh=mesh,
        out_type=jax.ShapeDtypeStruct((batch * HIST_PAD, dim), jnp.float32),
        scratch_types=[
            pltpu.VMEM((n_steps, CHUNK), jnp.int32),
        ]
        + [pltpu.VMEM((CHUNK, dim), jnp.float32)] * NBUF
        + [pltpu.SemaphoreType.DMA] * (2 * NBUF),
    )
    def emb_kernel(idx_hbm, table_hbm, out_hbm, idx_v, *rest):
        bufs = rest[:NBUF]
        gsem = rest[NBUF : 2 * NBUF]
        wsem = rest[2 * NBUF : 3 * NBUF]
        wid = lax.axis_index("s") * NUM_CORES + lax.axis_index("c")
        pltpu.sync_copy(idx_hbm.at[wid], idx_v)
        item_base = wid * items_per_w

        def fire_gather(j, b):
            pltpu.async_copy(table_hbm.at[idx_v.at[j]], bufs[b], gsem[b])

        def wait_gather(b):
            pltpu.make_async_copy(table_hbm.at[pl.ds(0, CHUNK)], bufs[b], gsem[b]).wait()

        def fire_write(j, b):
            item0 = item_base + ITEMS_PER_STEP * j
            pltpu.async_copy(
                bufs[b], out_hbm.at[pl.ds(item0 * HIST_PAD, CHUNK)], wsem[b]
            )

        def wait_write(b):
            pltpu.make_async_copy(bufs[b], out_hbm.at[pl.ds(0, CHUNK)], wsem[b]).wait()

        for j in range(NBUF):
            fire_gather(j, j)
            if j >= LAG:
                b2 = j - LAG
                wait_gather(b2)
                fire_write(b2, b2)

        def group(g, carry):
            j0 = g * NBUF
            for b in range(NBUF):
                wait_write(b)
                fire_gather(j0 + b, b)
                b2 = (b - LAG) % NBUF
                wait_gather(b2)
                fire_write(j0 + b - LAG, b2)
            return carry

        lax.fori_loop(1, n_groups, group, 0)

        for k in range(LAG):
            j = n_steps - LAG + k
            b2 = j % NBUF
            wait_gather(b2)
            fire_write(j, b2)
        for b in range(NBUF):
            wait_write(b)

    return emb_kernel


NUM_SLICES = 4


def kernel(x, weight):
    batch, hist = x.shape
    dim = weight.shape[1]
    xi = x.astype(jnp.int32)
    xpad = jnp.concatenate([xi, xi[:, : HIST_PAD - hist]], axis=1)
    sub = batch // NUM_SLICES
    emb = _make_kernel(sub, dim)
    acc = jnp.zeros((batch, hist, dim), jnp.float32)
    for s in range(NUM_SLICES):
        idx_s = xpad[s * sub : (s + 1) * sub].reshape(NUM_WORKERS, -1, CHUNK)
        o = emb(idx_s, weight)
        piece = o.reshape(sub, HIST_PAD, dim)[:, :hist, :]
        acc = lax.dynamic_update_slice(acc, piece, (s * sub, 0, 0))
    return acc


# speedup vs baseline: 2.5090x; 2.5090x over previous
import functools

import jax
import jax.numpy as jnp
from jax import lax
from jax.experimental import pallas as pl
from jax.experimental.pallas import tpu as pltpu
from jax.experimental.pallas import tpu_sc as plsc

EMBED_DIM = 128
NUM_CORES = 2
NUM_SUBCORES = 16
NUM_WORKERS = NUM_CORES * NUM_SUBCORES
CHUNK = 128
NBUF = 5
LAG = 2


@functools.lru_cache(maxsize=None)
def _make_kernel(total, dim):
    per_w = total // NUM_WORKERS
    n_steps = per_w // CHUNK
    assert n_steps % NBUF == 0 and n_steps >= 2 * NBUF
    n_groups = n_steps // NBUF
    mesh = plsc.VectorSubcoreMesh(core_axis_name="c", subcore_axis_name="s")

    @functools.partial(
        pl.kernel,
        mesh=mesh,
        out_type=jax.ShapeDtypeStruct((total, dim), jnp.float32),
        scratch_types=[
            pltpu.VMEM((n_steps, CHUNK), jnp.int32),
        ]
        + [pltpu.VMEM((CHUNK, dim), jnp.float32)] * NBUF
        + [pltpu.SemaphoreType.DMA] * (2 * NBUF),
    )
    def emb_kernel(idx_hbm, table_hbm, out_hbm, idx_v, *rest):
        bufs = rest[:NBUF]
        gsem = rest[NBUF : 2 * NBUF]
        wsem = rest[2 * NBUF : 3 * NBUF]
        wid = lax.axis_index("s") * NUM_CORES + lax.axis_index("c")
        pltpu.sync_copy(idx_hbm.at[wid], idx_v)
        base = wid * per_w

        def fire_gather(j, b):
            pltpu.async_copy(table_hbm.at[idx_v.at[j]], bufs[b], gsem[b])

        def wait_gather(b):
            pltpu.make_async_copy(table_hbm.at[pl.ds(0, CHUNK)], bufs[b], gsem[b]).wait()

        def fire_write(j, b):
            pltpu.async_copy(bufs[b], out_hbm.at[pl.ds(base + j * CHUNK, CHUNK)], wsem[b])

        def wait_write(b):
            pltpu.make_async_copy(bufs[b], out_hbm.at[pl.ds(0, CHUNK)], wsem[b]).wait()

        for j in range(NBUF):
            fire_gather(j, j)
            if j >= LAG:
                b2 = j - LAG
                wait_gather(b2)
                fire_write(b2, b2)

        def group(g, carry):
            j0 = g * NBUF
            for b in range(NBUF):
                wait_write(b)
                fire_gather(j0 + b, b)
                b2 = (b - LAG) % NBUF
                wait_gather(b2)
                fire_write(j0 + b - LAG, b2)
            return carry

        lax.fori_loop(1, n_groups, group, 0)

        for k in range(LAG):
            j = n_steps - LAG + k
            b2 = j % NBUF
            wait_gather(b2)
            fire_write(j, b2)
        for b in range(NBUF):
            wait_write(b)

    return emb_kernel


def kernel(x, weight):
    batch, hist = x.shape
    dim = weight.shape[1]
    xt = jnp.transpose(x).astype(jnp.int32)
    idx = xt.reshape(NUM_WORKERS, -1, CHUNK)
    out = _make_kernel(batch * hist, dim)(idx, weight)
    return jnp.transpose(out.reshape(hist, batch, dim), (1, 0, 2))


# speedup vs baseline: 2.5854x; 1.0304x over previous
import functools

import jax
import jax.numpy as jnp
from jax import lax
from jax.experimental import pallas as pl
from jax.experimental.pallas import tpu as pltpu
from jax.experimental.pallas import tpu_sc as plsc

EMBED_DIM = 128
NUM_CORES = 2
NUM_SUBCORES = 16
NUM_WORKERS = NUM_CORES * NUM_SUBCORES
CHUNK = 128
NBUF = 5
LAG = 2


@functools.lru_cache(maxsize=None)
def _make_kernel(hist, batch, dim):
    n_steps = hist
    assert batch == NUM_WORKERS * CHUNK
    assert n_steps % NBUF == 0 and n_steps >= 2 * NBUF
    n_groups = n_steps // NBUF
    mesh = plsc.VectorSubcoreMesh(core_axis_name="c", subcore_axis_name="s")

    @functools.partial(
        pl.kernel,
        mesh=mesh,
        out_type=jax.ShapeDtypeStruct((hist * batch, dim), jnp.float32),
        scratch_types=[
            pltpu.VMEM((n_steps, CHUNK), jnp.int32),
        ]
        + [pltpu.VMEM((CHUNK, dim), jnp.float32)] * NBUF
        + [pltpu.SemaphoreType.DMA] * (2 * NBUF),
    )
    def emb_kernel(idx_hbm, table_hbm, out_hbm, idx_v, *rest):
        bufs = rest[:NBUF]
        gsem = rest[NBUF : 2 * NBUF]
        wsem = rest[2 * NBUF : 3 * NBUF]
        wid = lax.axis_index("s") * NUM_CORES + lax.axis_index("c")
        pltpu.sync_copy(idx_hbm.at[:, pl.ds(wid * CHUNK, CHUNK)], idx_v)

        def fire_gather(j, b):
            pltpu.async_copy(table_hbm.at[idx_v.at[j]], bufs[b], gsem[b])

        def wait_gather(b):
            pltpu.make_async_copy(table_hbm.at[pl.ds(0, CHUNK)], bufs[b], gsem[b]).wait()

        def fire_write(j, b):
            pltpu.async_copy(
                bufs[b], out_hbm.at[pl.ds(j * batch + wid * CHUNK, CHUNK)], wsem[b]
            )

        def wait_write(b):
            pltpu.make_async_copy(bufs[b], out_hbm.at[pl.ds(0, CHUNK)], wsem[b]).wait()

        for j in range(NBUF):
            fire_gather(j, j)
            if j >= LAG:
                b2 = j - LAG
                wait_gather(b2)
                fire_write(b2, b2)

        def group(g, carry):
            j0 = g * NBUF
            for b in range(NBUF):
                wait_write(b)
                fire_gather(j0 + b, b)
                b2 = (b - LAG) % NBUF
                wait_gather(b2)
                fire_write(j0 + b - LAG, b2)
            return carry

        lax.fori_loop(1, n_groups, group, 0)

        for k in range(LAG):
            j = n_steps - LAG + k
            b2 = j % NBUF
            wait_gather(b2)
            fire_write(j, b2)
        for b in range(NBUF):
            wait_write(b)

    return emb_kernel


def kernel(x, weight):
    batch, hist = x.shape
    dim = weight.shape[1]
    xt = jnp.transpose(x).astype(jnp.int32)
    out = _make_kernel(hist, batch, dim)(xt, weight)
    return jnp.transpose(out.reshape(hist, batch, dim), (1, 0, 2))
